# Initial kernel scaffold; baseline (speedup 1.0000x reference)
import jax
import jax.numpy as jnp
from jax.experimental import pallas as pl


def kernel(token_ids, token_embedding, position_embedding):
    raise NotImplementedError("write your pallas kernel here")


# speedup vs baseline: 1.2439x; 1.2439x over previous
import functools

import jax
import jax.numpy as jnp
from jax import lax
from jax.experimental import pallas as pl
from jax.experimental.pallas import tpu as pltpu
from jax.experimental.pallas import tpu_sc as plsc

_VOCAB = 100000
_SEQ = 2048
_BATCH = 4
_D = 128
_ROWS = _BATCH * _SEQ
_NC = 2
_NS = 16
_NW = _NC * _NS
_BPW = _ROWS // _NW
_CH = 128
_NCH = _BPW // _CH
_L = 16


def _emb_body(ids_hbm, pos_hbm, tab_hbm, out_hbm, idx_v, rows_v, pos_v, sem):
    wid = lax.axis_index("s") * _NC + lax.axis_index("c")
    base = wid * _BPW
    pos_base = lax.rem(base, _SEQ)

    for j in range(_NCH):
        pltpu.sync_copy(ids_hbm.at[pl.ds(base + j * _CH, _CH)], idx_v.at[j])

    descs = [
        pltpu.async_copy(tab_hbm.at[idx_v.at[j]], rows_v.at[j], sem)
        for j in range(_NCH)
    ]
    pltpu.sync_copy(pos_hbm.at[pl.ds(pos_base, _BPW)], pos_v)
    for d in descs:
        d.wait()

    for j in range(_NCH):
        def add_row(r, _, j=j):
            for c in range(_D // _L):
                sl = pl.ds(c * _L, _L)
                rows_v[j, r, sl] = rows_v[j, r, sl] + pos_v[j * _CH + r, sl]
            return _
        lax.fori_loop(0, _CH, add_row, 0)
        pltpu.sync_copy(rows_v.at[j],
                        out_hbm.at[pl.ds(base + j * _CH, _CH)])


@jax.jit
def _emb_call(ids_flat, token_embedding, position_embedding):
    mesh = plsc.VectorSubcoreMesh(core_axis_name="c", subcore_axis_name="s")
    run = pl.kernel(
        _emb_body,
        out_type=jax.ShapeDtypeStruct((_ROWS, _D), jnp.float32),
        mesh=mesh,
        scratch_types=[
            pltpu.VMEM((_NCH, _CH), jnp.int32),
            pltpu.VMEM((_NCH, _CH, _D), jnp.float32),
            pltpu.VMEM((_BPW, _D), jnp.float32),
            pltpu.SemaphoreType.DMA,
        ],
    )
    return run(ids_flat, position_embedding, token_embedding)


def kernel(token_ids, token_embedding, position_embedding):
    ids_flat = jnp.reshape(token_ids, (_ROWS,)).astype(jnp.int32)
    out = _emb_call(ids_flat, token_embedding, position_embedding)
    return jnp.reshape(out, (_BATCH, _SEQ, _D))


# speedup vs baseline: 1.3282x; 1.0678x over previous
import functools

import jax
import jax.numpy as jnp
from jax import lax
from jax.experimental import pallas as pl
from jax.experimental.pallas import tpu as pltpu
from jax.experimental.pallas import tpu_sc as plsc

_VOCAB = 100000
_SEQ = 2048
_BATCH = 4
_D = 128
_ROWS = _BATCH * _SEQ
_NC = 2
_NS = 16
_NW = _NC * _NS
_BPW = _ROWS // _NW
_CH = 128
_NCH = _BPW // _CH
_L = 16


def _emb_body(ids_hbm, pos_hbm, tab_hbm, out_hbm, idx_v, rows_v, pos_v,
              psem, gsems, osems):
    wid = lax.axis_index("s") * _NC + lax.axis_index("c")
    base = wid * _BPW
    pos_base = lax.rem(base, _SEQ)

    pdesc = pltpu.async_copy(pos_hbm.at[pl.ds(pos_base, _BPW)], pos_v, psem)

    gdescs = []
    for j in range(_NCH):
        pltpu.sync_copy(ids_hbm.at[pl.ds(base + j * _CH, _CH)], idx_v.at[j])
        gdescs.append(
            pltpu.async_copy(tab_hbm.at[idx_v.at[j]], rows_v.at[j],
                             gsems.at[j]))
    pdesc.wait()

    odescs = []
    for j in range(_NCH):
        gdescs[j].wait()

        def add_row(r, _, j=j):
            for c in range(_D // _L):
                sl = pl.ds(c * _L, _L)
                plsc.addupdate(rows_v.at[j, r, sl], pos_v[j * _CH + r, sl])
            return _
        lax.fori_loop(0, _CH, add_row, 0)
        odescs.append(
            pltpu.async_copy(rows_v.at[j],
                             out_hbm.at[pl.ds(base + j * _CH, _CH)],
                             osems.at[j]))
    for d in odescs:
        d.wait()


@jax.jit
def _emb_call(ids_flat, token_embedding, position_embedding):
    mesh = plsc.VectorSubcoreMesh(core_axis_name="c", subcore_axis_name="s")
    run = pl.kernel(
        _emb_body,
        out_type=jax.ShapeDtypeStruct((_ROWS, _D), jnp.float32),
        mesh=mesh,
        scratch_types=[
            pltpu.VMEM((_NCH, _CH), jnp.int32),
            pltpu.VMEM((_NCH, _CH, _D), jnp.float32),
            pltpu.VMEM((_BPW, _D), jnp.float32),
            pltpu.SemaphoreType.DMA,
            pltpu.SemaphoreType.DMA((_NCH,)),
            pltpu.SemaphoreType.DMA((_NCH,)),
        ],
    )
    return run(ids_flat, position_embedding, token_embedding)


def kernel(token_ids, token_embedding, position_embedding):
    ids_flat = jnp.reshape(token_ids, (_ROWS,)).astype(jnp.int32)
    out = _emb_call(ids_flat, token_embedding, position_embedding)
    return jnp.reshape(out, (_BATCH, _SEQ, _D))


# speedup vs baseline: 1.4246x; 1.0726x over previous
import functools

import jax
import jax.numpy as jnp
from jax import lax
from jax.experimental import pallas as pl
from jax.experimental.pallas import tpu as pltpu
from jax.experimental.pallas import tpu_sc as plsc

_VOCAB = 100000
_SEQ = 2048
_BATCH = 4
_D = 128
_ROWS = _BATCH * _SEQ
_NC = 2
_NS = 16
_NW = _NC * _NS
_PPW = _SEQ // _NW
_CH = _PPW
_NCH = _BATCH
_L = 16


def _emb_body(ids_hbm, pos_hbm, tab_hbm, out_hbm, idx_v, rows_v, pos_v,
              psem, gsems, osems):
    wid = lax.axis_index("s") * _NC + lax.axis_index("c")
    pos_base = wid * _PPW

    pdesc = pltpu.async_copy(pos_hbm.at[pl.ds(pos_base, _PPW)], pos_v, psem)

    gdescs = []
    for j in range(_NCH):
        row0 = j * _SEQ + pos_base
        pltpu.sync_copy(ids_hbm.at[pl.ds(row0, _CH)], idx_v.at[j])
        gdescs.append(
            pltpu.async_copy(tab_hbm.at[idx_v.at[j]], rows_v.at[j],
                             gsems.at[j]))
    pdesc.wait()

    odescs = []
    for j in range(_NCH):
        gdescs[j].wait()

        def add_row(r, _, j=j):
            for c in range(_D // _L):
                sl = pl.ds(c * _L, _L)
                plsc.addupdate(rows_v.at[j, r, sl], pos_v[r, sl])
            return _
        lax.fori_loop(0, _CH, add_row, 0)
        odescs.append(
            pltpu.async_copy(rows_v.at[j],
                             out_hbm.at[pl.ds(j * _SEQ + pos_base, _CH)],
                             osems.at[j]))
    for d in odescs:
        d.wait()


@jax.jit
def _emb_call(ids_flat, token_embedding, position_embedding):
    mesh = plsc.VectorSubcoreMesh(core_axis_name="c", subcore_axis_name="s")
    run = pl.kernel(
        _emb_body,
        out_type=jax.ShapeDtypeStruct((_ROWS, _D), jnp.float32),
        mesh=mesh,
        scratch_types=[
            pltpu.VMEM((_NCH, _CH), jnp.int32),
            pltpu.VMEM((_NCH, _CH, _D), jnp.float32),
            pltpu.VMEM((_PPW, _D), jnp.float32),
            pltpu.SemaphoreType.DMA,
            pltpu.SemaphoreType.DMA((_NCH,)),
            pltpu.SemaphoreType.DMA((_NCH,)),
        ],
    )
    return run(ids_flat, position_embedding, token_embedding)


def kernel(token_ids, token_embedding, position_embedding):
    ids_flat = jnp.reshape(token_ids, (_ROWS,)).astype(jnp.int32)
    out = _emb_call(ids_flat, token_embedding, position_embedding)
    return jnp.reshape(out, (_BATCH, _SEQ, _D))
